# baseline (speedup 1.0000x reference)
import functools

import jax
import jax.numpy as jnp
import numpy as np
from jax import lax
from jax.experimental import pallas as pl
from jax.experimental.pallas import tpu as pltpu
from jax.experimental.pallas import tpu_sc as plsc

N = 10000
NPAD = 10240
E_REAL = 330000
CHUNK = 128
NCORES = 2
NSUB = 16
CH_PER_TILE = 81
EPAD = NCORES * NSUB * CH_PER_TILE * CHUNK
STRIPE = NPAD // NSUB
BN = 1280
GRID = NPAD // BN
ROWS_PER_TILE = CH_PER_TILE


def _emit_outputs(i, acc, h_ref, asad_ref, adas_ref, mv_ref):
    d = h_ref.shape[-1]
    h_ref[...] = acc[:, :d]
    asad = acc[:, d:]
    asad_ref[...] = asad
    adas_ref[...] = jnp.concatenate([acc[:, d + 8:], acc[:, d:d + 8]], axis=1)
    bmax = jnp.max(asad, axis=0, keepdims=True)
    bmax = jnp.broadcast_to(bmax, (8, 16))

    @pl.when(i == 0)
    def _():
        mv_ref[...] = bmax

    @pl.when(i > 0)
    def _():
        mv_ref[...] = jnp.maximum(mv_ref[...], bmax)


def _mm_first_body(z_ref, wc_ref, h_ref, asad_ref, adas_ref, mv_ref):
    i = pl.program_id(0)
    acc = jnp.dot(z_ref[...], wc_ref[...], preferred_element_type=jnp.float32)
    _emit_outputs(i, acc, h_ref, asad_ref, adas_ref, mv_ref)


def _mm_next_body(hc, cc, p_ref, b_ref, wc_ref, h_ref, asad_ref, adas_ref,
                  mv_ref):
    i = pl.program_id(0)
    p = p_ref[...]
    d_in = hc * cc
    msg = p[0, :, :d_in] + p[1, :, :d_in]
    den8 = p[0, :, d_in:d_in + 8] + p[1, :, d_in:d_in + 8]
    den = jnp.repeat(den8, cc, axis=1)
    z = msg / (den + 1e-16) + b_ref[...]
    z = jnp.where(z > 0, z, jnp.exp(z) - 1.0)
    acc = jnp.dot(z, wc_ref[...], preferred_element_type=jnp.float32)
    _emit_outputs(i, acc, h_ref, asad_ref, adas_ref, mv_ref)


def _mm_outs(d_out):
    return dict(
        out_specs=[
            pl.BlockSpec((BN, d_out), lambda i: (i, 0)),
            pl.BlockSpec((BN, 16), lambda i: (i, 0)),
            pl.BlockSpec((BN, 16), lambda i: (i, 0)),
            pl.BlockSpec((8, 16), lambda i: (0, 0)),
        ],
        out_shape=[
            jax.ShapeDtypeStruct((NPAD, d_out), jnp.float32),
            jax.ShapeDtypeStruct((NPAD, 16), jnp.float32),
            jax.ShapeDtypeStruct((NPAD, 16), jnp.float32),
            jax.ShapeDtypeStruct((8, 16), jnp.float32),
        ],
    )


def _tc_matmul_first(x_pad, wc, d_out):
    return pl.pallas_call(
        _mm_first_body,
        grid=(GRID,),
        in_specs=[
            pl.BlockSpec((BN, x_pad.shape[1]), lambda i: (i, 0)),
            pl.BlockSpec(wc.shape, lambda i: (0, 0)),
        ],
        **_mm_outs(d_out),
    )(x_pad, wc)


def _tc_matmul_next(parts, bvec, wc, d_out, hc, cc):
    d_in = parts.shape[-1]
    return pl.pallas_call(
        functools.partial(_mm_next_body, hc, cc),
        grid=(GRID,),
        in_specs=[
            pl.BlockSpec((2, BN, d_in), lambda i: (0, i, 0)),
            pl.BlockSpec((1, hc * cc), lambda i: (0, 0)),
            pl.BlockSpec(wc.shape, lambda i: (0, 0)),
        ],
        **_mm_outs(d_out),
    )(parts, bvec, wc)


def _final_body(p_ref, b_ref, out_ref):
    p = p_ref[...]
    msg = p[0, :, :16] + p[1, :, :16]
    den = p[0, :, 16:17] + p[1, :, 16:17]
    z = msg / (den + 1e-16) + b_ref[...]
    m = jnp.max(z, axis=1, keepdims=True)
    s = jnp.log(jnp.sum(jnp.exp(z - m), axis=1, keepdims=True))
    out_ref[...] = z - m - s


def _tc_final(parts, bvec):
    d_in = parts.shape[-1]
    return pl.pallas_call(
        _final_body,
        grid=(GRID,),
        in_specs=[
            pl.BlockSpec((2, BN, d_in), lambda i: (0, i, 0)),
            pl.BlockSpec((1, 16), lambda i: (0, 0)),
        ],
        out_specs=pl.BlockSpec((BN, 16), lambda i: (i, 0)),
        out_shape=jax.ShapeDtypeStruct((NPAD, 16), jnp.float32),
    )(parts, bvec)


def _vtake(v, idx):
    dn = lax.GatherDimensionNumbers(
        offset_dims=(), collapsed_slice_dims=(0,), start_index_map=(0,))
    return lax.gather(v, idx[:, None], dn, (1,),
                      mode=lax.GatherScatterMode.PROMISE_IN_BOUNDS)


@functools.cache
def _make_edge(d_feat, c_per_head):
    kv = d_feat // 16
    kw = kv + 1
    mesh = plsc.VectorSubcoreMesh(core_axis_name="c", subcore_axis_name="s")

    @functools.partial(
        pl.kernel,
        mesh=mesh,
        compiler_params=pltpu.CompilerParams(
            needs_layout_passes=False, use_tc_tiling_on_sc=False),
        out_type=jax.ShapeDtypeStruct((NCORES, NPAD, kw, 16), jnp.float32),
        scratch_types=[
            pltpu.VMEM_SHARED((NPAD, kw, 16), jnp.float32),
            pltpu.VMEM((ROWS_PER_TILE * CHUNK,), jnp.int32),
            pltpu.VMEM((ROWS_PER_TILE * CHUNK,), jnp.int32),
            pltpu.VMEM((CHUNK, 16), jnp.float32),
            pltpu.VMEM((CHUNK, 16), jnp.float32),
            pltpu.VMEM((CHUNK, 16), jnp.float32),
            pltpu.VMEM((CHUNK, 16), jnp.float32),
            pltpu.VMEM((CHUNK, kv, 16), jnp.float32),
            pltpu.VMEM((CHUNK, kv, 16), jnp.float32),
            pltpu.VMEM((CHUNK, kw, 16), jnp.float32),
            pltpu.VMEM((CHUNK, kw, 16), jnp.float32),
            pltpu.VMEM((16,), jnp.float32),
            pltpu.VMEM((64, kw, 16), jnp.float32),
            pltpu.SemaphoreType.DMA,
            pltpu.SemaphoreType.DMA,
            pltpu.SemaphoreType.DMA,
            pltpu.SemaphoreType.DMA,
            pltpu.SemaphoreType.DMA,
            pltpu.SemaphoreType.DMA,
            pltpu.SemaphoreType.DMA,
            pltpu.SemaphoreType.DMA,
        ],
    )
    def edge(asad, adas, hrows, srcp, dstp, mvec, out_hbm,
             sh_out, isrc, idst, srow0, drow0, srow1, drow1,
             hbuf0, hbuf1, msgbuf0, msgbuf1, m_v, zbuf,
             ss0, sd0, sh0, ss1, sd1, sh1, sc0, sc1):
        c = lax.axis_index("c")
        s = lax.axis_index("s")

        @plsc.parallel_loop(0, 64, unroll=4)
        def _zb(i):
            for k in range(kw):
                zbuf[i, k] = jnp.zeros((16,), jnp.float32)

        for seg in range(STRIPE // 64):
            pltpu.sync_copy(zbuf,
                            sh_out.at[pl.ds(s * STRIPE + seg * 64, 64)])
        tbase = c * (EPAD // 2) + s * (ROWS_PER_TILE * CHUNK)
        pltpu.sync_copy(srcp.at[pl.ds(tbase, ROWS_PER_TILE * CHUNK)], isrc)
        pltpu.sync_copy(dstp.at[pl.ds(tbase, ROWS_PER_TILE * CHUNK)], idst)
        pltpu.sync_copy(mvec, m_v)
        plsc.subcore_barrier()
        lane = lax.iota(jnp.int32, 16)
        rot8 = (lane + 8) & 15
        mrow = m_v[...]
        msum = mrow + _vtake(mrow, rot8)
        m16 = jnp.where(msum > 0, msum, 0.2 * msum)
        if c_per_head == 8:
            exp_idx = [lax.shift_right_logical(lane, 3) + 2 * k
                       for k in range(kv)]
        else:
            exp_idx = [lane * 0 + k for k in range(kv)]

        def start(n, srow, drow, hbuf, ss, sd, sh):
            ixs = isrc.at[pl.ds(n * CHUNK, CHUNK)]
            ixd = idst.at[pl.ds(n * CHUNK, CHUNK)]
            pltpu.async_copy(asad.at[ixs], srow, ss)
            pltpu.async_copy(adas.at[ixd], drow, sd)
            pltpu.async_copy(hrows.at[ixs], hbuf, sh)

        def waitg(srow, drow, hbuf, ss, sd, sh):
            ix0 = isrc.at[pl.ds(0, CHUNK)]
            pltpu.make_async_copy(asad.at[ix0], srow, ss).wait()
            pltpu.make_async_copy(adas.at[ix0], drow, sd).wait()
            pltpu.make_async_copy(hrows.at[ix0], hbuf, sh).wait()

        def wait_scatter(msgbuf, sc):
            pltpu.make_async_copy(
                msgbuf, sh_out.at[idst.at[pl.ds(0, CHUNK)]], sc).wait()

        def compute(n, srow, drow, hbuf, msgbuf, sc):
            @plsc.parallel_loop(0, CHUNK, unroll=2)
            def _eb(j):
                q = srow[j] + drow[j]
                e = jnp.where(q > 0, q, 0.2 * q)
                ex = jnp.exp(e - m16)
                msgbuf[j, kv] = ex
                for k in range(kv):
                    av = _vtake(ex, exp_idx[k])
                    msgbuf[j, k] = av * hbuf[j, k]

            pltpu.async_copy(
                msgbuf, sh_out.at[idst.at[pl.ds(n * CHUNK, CHUNK)]], sc,
                add=True)

        start(0, srow0, drow0, hbuf0, ss0, sd0, sh0)

        def body(i, carry):
            a = 2 * i
            start(a + 1, srow1, drow1, hbuf1, ss1, sd1, sh1)
            waitg(srow0, drow0, hbuf0, ss0, sd0, sh0)

            @pl.when(i > 0)
            def _():
                wait_scatter(msgbuf0, sc0)

            compute(a, srow0, drow0, hbuf0, msgbuf0, sc0)
            start(a + 2, srow0, drow0, hbuf0, ss0, sd0, sh0)
            waitg(srow1, drow1, hbuf1, ss1, sd1, sh1)

            @pl.when(i > 0)
            def _():
                wait_scatter(msgbuf1, sc1)

            compute(a + 1, srow1, drow1, hbuf1, msgbuf1, sc1)
            return carry

        lax.fori_loop(0, (ROWS_PER_TILE - 1) // 2, body, 0)
        waitg(srow0, drow0, hbuf0, ss0, sd0, sh0)
        wait_scatter(msgbuf0, sc0)
        compute(ROWS_PER_TILE - 1, srow0, drow0, hbuf0, msgbuf0, sc0)
        wait_scatter(msgbuf0, sc0)
        wait_scatter(msgbuf1, sc1)

        plsc.subcore_barrier()
        pltpu.sync_copy(sh_out.at[pl.ds(s * STRIPE, STRIPE)],
                        out_hbm.at[c, pl.ds(s * STRIPE, STRIPE)])

    return edge


def _fold_wc(W, a_s, a_d, H, C):
    d_in = W.shape[0]
    Wr = W.reshape(d_in, H, C)
    was = (Wr * a_s[None]).sum(-1)
    wad = (Wr * a_d[None]).sum(-1)
    if H == 8:
        return jnp.concatenate([W, was, wad], axis=1)
    z7 = jnp.zeros((d_in, 7), jnp.float32)
    return jnp.concatenate([W, was, z7, wad, z7], axis=1)


def _edge_phase(h, asad, adas, mv, srcf, dstf, C, D):
    kv = D // 16
    h3d = h.reshape(NPAD, kv, 16)
    outp = _make_edge(D, C)(asad, adas, h3d, srcf, dstf, mv[0])
    return outp.reshape(NCORES, NPAD, (kv + 1) * 16)


def kernel(x, edge_index, W1, a_s1, a_d1, b1, W2, a_s2, a_d2, b2,
           W3, a_s3, a_d3, b3):
    loop = jnp.arange(N, dtype=edge_index.dtype)
    ei = jnp.concatenate([edge_index, jnp.stack([loop, loop])], axis=1)
    ei = jnp.pad(ei, ((0, 0), (0, EPAD - E_REAL)), constant_values=N)
    src_f = ei[0]
    dst_f = ei[1]
    x_pad = jnp.pad(x, ((0, NPAD - N), (0, 0)))

    wc1 = _fold_wc(W1, a_s1, a_d1, 8, 8)
    wc2 = _fold_wc(W2, a_s2, a_d2, 8, 8)
    wc3 = _fold_wc(W3, a_s3, a_d3, 1, 16)

    h1, asad1, adas1, mv1 = _tc_matmul_first(x_pad, wc1, 64)
    outp1 = _edge_phase(h1, asad1, adas1, mv1, src_f, dst_f, 8, 64)

    h2, asad2, adas2, mv2 = _tc_matmul_next(
        outp1, b1.reshape(1, 64), wc2, 64, 8, 8)
    outp2 = _edge_phase(h2, asad2, adas2, mv2, src_f, dst_f, 8, 64)

    h3, asad3, adas3, mv3 = _tc_matmul_next(
        outp2, b2.reshape(1, 64), wc3, 16, 8, 8)
    outp3 = _edge_phase(h3, asad3, adas3, mv3, src_f, dst_f, 16, 16)

    out = _tc_final(outp3, b3.reshape(1, 16))
    return out[:N]


# speedup vs baseline: 1.0019x; 1.0019x over previous
import functools

import jax
import jax.numpy as jnp
import numpy as np
from jax import lax
from jax.experimental import pallas as pl
from jax.experimental.pallas import tpu as pltpu
from jax.experimental.pallas import tpu_sc as plsc

N = 10000
NPAD = 10240
E_REAL = 330000
CHUNK = 128
NCORES = 2
NSUB = 16
CH_PER_TILE = 81
EPAD = NCORES * NSUB * CH_PER_TILE * CHUNK
STRIPE = NPAD // NSUB
BN = 1280
GRID = NPAD // BN
ROWS_PER_TILE = CH_PER_TILE


def _emit_outputs(i, acc, h_ref, asad_ref, adas_ref, mv_ref):
    d = h_ref.shape[-1]
    h_ref[...] = acc[:, :d]
    asad = acc[:, d:]
    asad_ref[...] = asad
    adas_ref[...] = jnp.concatenate([acc[:, d + 8:], acc[:, d:d + 8]], axis=1)
    bmax = jnp.max(asad, axis=0, keepdims=True)
    bmax = jnp.broadcast_to(bmax, (8, 16))

    @pl.when(i == 0)
    def _():
        mv_ref[...] = bmax

    @pl.when(i > 0)
    def _():
        mv_ref[...] = jnp.maximum(mv_ref[...], bmax)


def _mm_first_body(z_ref, wc_ref, h_ref, asad_ref, adas_ref, mv_ref):
    i = pl.program_id(0)
    acc = jnp.dot(z_ref[...], wc_ref[...], preferred_element_type=jnp.float32)
    _emit_outputs(i, acc, h_ref, asad_ref, adas_ref, mv_ref)


def _mm_next_body(hc, cc, p_ref, b_ref, wc_ref, h_ref, asad_ref, adas_ref,
                  mv_ref):
    i = pl.program_id(0)
    p = p_ref[...]
    d_in = hc * cc
    msg = p[0, :, :d_in] + p[1, :, :d_in]
    den8 = p[0, :, d_in:d_in + 8] + p[1, :, d_in:d_in + 8]
    den = jnp.repeat(den8, cc, axis=1)
    z = msg / (den + 1e-16) + b_ref[...]
    z = jnp.where(z > 0, z, jnp.exp(z) - 1.0)
    acc = jnp.dot(z, wc_ref[...], preferred_element_type=jnp.float32)
    _emit_outputs(i, acc, h_ref, asad_ref, adas_ref, mv_ref)


def _mm_outs(d_out):
    return dict(
        out_specs=[
            pl.BlockSpec((BN, d_out), lambda i: (i, 0)),
            pl.BlockSpec((BN, 16), lambda i: (i, 0)),
            pl.BlockSpec((BN, 16), lambda i: (i, 0)),
            pl.BlockSpec((8, 16), lambda i: (0, 0)),
        ],
        out_shape=[
            jax.ShapeDtypeStruct((NPAD, d_out), jnp.float32),
            jax.ShapeDtypeStruct((NPAD, 16), jnp.float32),
            jax.ShapeDtypeStruct((NPAD, 16), jnp.float32),
            jax.ShapeDtypeStruct((8, 16), jnp.float32),
        ],
    )


def _tc_matmul_first(x_pad, wc, d_out):
    return pl.pallas_call(
        _mm_first_body,
        grid=(GRID,),
        in_specs=[
            pl.BlockSpec((BN, x_pad.shape[1]), lambda i: (i, 0)),
            pl.BlockSpec(wc.shape, lambda i: (0, 0)),
        ],
        **_mm_outs(d_out),
    )(x_pad, wc)


def _tc_matmul_next(parts, bvec, wc, d_out, hc, cc):
    d_in = parts.shape[-1]
    return pl.pallas_call(
        functools.partial(_mm_next_body, hc, cc),
        grid=(GRID,),
        in_specs=[
            pl.BlockSpec((2, BN, d_in), lambda i: (0, i, 0)),
            pl.BlockSpec((1, hc * cc), lambda i: (0, 0)),
            pl.BlockSpec(wc.shape, lambda i: (0, 0)),
        ],
        **_mm_outs(d_out),
    )(parts, bvec, wc)


def _final_body(p_ref, b_ref, out_ref):
    p = p_ref[...]
    msg = p[0, :, :16] + p[1, :, :16]
    den = p[0, :, 16:17] + p[1, :, 16:17]
    z = msg / (den + 1e-16) + b_ref[...]
    m = jnp.max(z, axis=1, keepdims=True)
    s = jnp.log(jnp.sum(jnp.exp(z - m), axis=1, keepdims=True))
    out_ref[...] = z - m - s


def _tc_final(parts, bvec):
    d_in = parts.shape[-1]
    return pl.pallas_call(
        _final_body,
        grid=(GRID,),
        in_specs=[
            pl.BlockSpec((2, BN, d_in), lambda i: (0, i, 0)),
            pl.BlockSpec((1, 16), lambda i: (0, 0)),
        ],
        out_specs=pl.BlockSpec((BN, 16), lambda i: (i, 0)),
        out_shape=jax.ShapeDtypeStruct((NPAD, 16), jnp.float32),
    )(parts, bvec)


def _vtake(v, idx):
    dn = lax.GatherDimensionNumbers(
        offset_dims=(), collapsed_slice_dims=(0,), start_index_map=(0,))
    return lax.gather(v, idx[:, None], dn, (1,),
                      mode=lax.GatherScatterMode.PROMISE_IN_BOUNDS)


@functools.cache
def _make_edge(d_feat, c_per_head):
    kv = d_feat // 16
    kw = kv + 1
    mesh = plsc.VectorSubcoreMesh(core_axis_name="c", subcore_axis_name="s")

    @functools.partial(
        pl.kernel,
        mesh=mesh,
        compiler_params=pltpu.CompilerParams(
            needs_layout_passes=False, use_tc_tiling_on_sc=False),
        out_type=jax.ShapeDtypeStruct((NCORES, NPAD, kw, 16), jnp.float32),
        scratch_types=[
            pltpu.VMEM_SHARED((NPAD, kw, 16), jnp.float32),
            pltpu.VMEM((ROWS_PER_TILE * CHUNK,), jnp.int32),
            pltpu.VMEM((ROWS_PER_TILE * CHUNK,), jnp.int32),
            pltpu.VMEM((CHUNK, 16), jnp.float32),
            pltpu.VMEM((CHUNK, 16), jnp.float32),
            pltpu.VMEM((CHUNK, 16), jnp.float32),
            pltpu.VMEM((CHUNK, 16), jnp.float32),
            pltpu.VMEM((CHUNK, kv, 16), jnp.float32),
            pltpu.VMEM((CHUNK, kv, 16), jnp.float32),
            pltpu.VMEM((CHUNK, kw, 16), jnp.float32),
            pltpu.VMEM((CHUNK, kw, 16), jnp.float32),
            pltpu.VMEM((16,), jnp.float32),
            pltpu.VMEM((64, kw, 16), jnp.float32),
            pltpu.SemaphoreType.DMA,
            pltpu.SemaphoreType.DMA,
            pltpu.SemaphoreType.DMA,
            pltpu.SemaphoreType.DMA,
            pltpu.SemaphoreType.DMA,
            pltpu.SemaphoreType.DMA,
            pltpu.SemaphoreType.DMA,
            pltpu.SemaphoreType.DMA,
        ],
    )
    def edge(asad, adas, hrows, srcp, dstp, mvec, out_hbm,
             sh_out, isrc, idst, srow0, drow0, srow1, drow1,
             hbuf0, hbuf1, msgbuf0, msgbuf1, m_v, zbuf,
             ss0, sd0, sh0, ss1, sd1, sh1, sc0, sc1):
        c = lax.axis_index("c")
        s = lax.axis_index("s")

        @plsc.parallel_loop(0, 64, unroll=4)
        def _zb(i):
            for k in range(kw):
                zbuf[i, k] = jnp.zeros((16,), jnp.float32)

        for seg in range(STRIPE // 64):
            pltpu.sync_copy(zbuf,
                            sh_out.at[pl.ds(s * STRIPE + seg * 64, 64)])
        tbase = c * (EPAD // 2) + s * (ROWS_PER_TILE * CHUNK)
        pltpu.sync_copy(srcp.at[pl.ds(tbase, ROWS_PER_TILE * CHUNK)], isrc)
        pltpu.sync_copy(dstp.at[pl.ds(tbase, ROWS_PER_TILE * CHUNK)], idst)
        pltpu.sync_copy(mvec, m_v)
        plsc.subcore_barrier()
        lane = lax.iota(jnp.int32, 16)
        rot8 = (lane + 8) & 15
        mrow = m_v[...]
        msum = mrow + _vtake(mrow, rot8)
        m16 = jnp.where(msum > 0, msum, 0.2 * msum)
        if c_per_head == 8:
            exp_idx = [lax.shift_right_logical(lane, 3) + 2 * k
                       for k in range(kv)]
        else:
            exp_idx = [lane * 0 + k for k in range(kv)]

        def start(n, srow, drow, hbuf, ss, sd, sh):
            ixs = isrc.at[pl.ds(n * CHUNK, CHUNK)]
            ixd = idst.at[pl.ds(n * CHUNK, CHUNK)]
            pltpu.async_copy(asad.at[ixs], srow, ss)
            pltpu.async_copy(adas.at[ixd], drow, sd)
            pltpu.async_copy(hrows.at[ixs], hbuf, sh)

        def waitg(srow, drow, hbuf, ss, sd, sh):
            ix0 = isrc.at[pl.ds(0, CHUNK)]
            pltpu.make_async_copy(asad.at[ix0], srow, ss).wait()
            pltpu.make_async_copy(adas.at[ix0], drow, sd).wait()
            pltpu.make_async_copy(hrows.at[ix0], hbuf, sh).wait()

        def wait_scatter(msgbuf, sc):
            pltpu.make_async_copy(
                msgbuf, sh_out.at[idst.at[pl.ds(0, CHUNK)]], sc).wait()

        def compute(n, srow, drow, hbuf, msgbuf, sc):
            @plsc.parallel_loop(0, CHUNK, unroll=4)
            def _eb(j):
                q = srow[j] + drow[j]
                e = jnp.where(q > 0, q, 0.2 * q)
                ex = jnp.exp(e - m16)
                msgbuf[j, kv] = ex
                for k in range(kv):
                    av = _vtake(ex, exp_idx[k])
                    msgbuf[j, k] = av * hbuf[j, k]

            pltpu.async_copy(
                msgbuf, sh_out.at[idst.at[pl.ds(n * CHUNK, CHUNK)]], sc,
                add=True)

        start(0, srow0, drow0, hbuf0, ss0, sd0, sh0)

        def body(i, carry):
            a = 2 * i
            start(a + 1, srow1, drow1, hbuf1, ss1, sd1, sh1)
            waitg(srow0, drow0, hbuf0, ss0, sd0, sh0)

            @pl.when(i > 0)
            def _():
                wait_scatter(msgbuf0, sc0)

            compute(a, srow0, drow0, hbuf0, msgbuf0, sc0)
            start(a + 2, srow0, drow0, hbuf0, ss0, sd0, sh0)
            waitg(srow1, drow1, hbuf1, ss1, sd1, sh1)

            @pl.when(i > 0)
            def _():
                wait_scatter(msgbuf1, sc1)

            compute(a + 1, srow1, drow1, hbuf1, msgbuf1, sc1)
            return carry

        lax.fori_loop(0, (ROWS_PER_TILE - 1) // 2, body, 0)
        waitg(srow0, drow0, hbuf0, ss0, sd0, sh0)
        wait_scatter(msgbuf0, sc0)
        compute(ROWS_PER_TILE - 1, srow0, drow0, hbuf0, msgbuf0, sc0)
        wait_scatter(msgbuf0, sc0)
        wait_scatter(msgbuf1, sc1)

        plsc.subcore_barrier()
        pltpu.sync_copy(sh_out.at[pl.ds(s * STRIPE, STRIPE)],
                        out_hbm.at[c, pl.ds(s * STRIPE, STRIPE)])

    return edge


def _fold_wc(W, a_s, a_d, H, C):
    d_in = W.shape[0]
    Wr = W.reshape(d_in, H, C)
    was = (Wr * a_s[None]).sum(-1)
    wad = (Wr * a_d[None]).sum(-1)
    if H == 8:
        return jnp.concatenate([W, was, wad], axis=1)
    z7 = jnp.zeros((d_in, 7), jnp.float32)
    return jnp.concatenate([W, was, z7, wad, z7], axis=1)


def _edge_phase(h, asad, adas, mv, srcf, dstf, C, D):
    kv = D // 16
    h3d = h.reshape(NPAD, kv, 16)
    outp = _make_edge(D, C)(asad, adas, h3d, srcf, dstf, mv[0])
    return outp.reshape(NCORES, NPAD, (kv + 1) * 16)


def kernel(x, edge_index, W1, a_s1, a_d1, b1, W2, a_s2, a_d2, b2,
           W3, a_s3, a_d3, b3):
    loop = jnp.arange(N, dtype=edge_index.dtype)
    ei = jnp.concatenate([edge_index, jnp.stack([loop, loop])], axis=1)
    ei = jnp.pad(ei, ((0, 0), (0, EPAD - E_REAL)), constant_values=N)
    src_f = ei[0]
    dst_f = ei[1]
    x_pad = jnp.pad(x, ((0, NPAD - N), (0, 0)))

    wc1 = _fold_wc(W1, a_s1, a_d1, 8, 8)
    wc2 = _fold_wc(W2, a_s2, a_d2, 8, 8)
    wc3 = _fold_wc(W3, a_s3, a_d3, 1, 16)

    h1, asad1, adas1, mv1 = _tc_matmul_first(x_pad, wc1, 64)
    outp1 = _edge_phase(h1, asad1, adas1, mv1, src_f, dst_f, 8, 64)

    h2, asad2, adas2, mv2 = _tc_matmul_next(
        outp1, b1.reshape(1, 64), wc2, 64, 8, 8)
    outp2 = _edge_phase(h2, asad2, adas2, mv2, src_f, dst_f, 8, 64)

    h3, asad3, adas3, mv3 = _tc_matmul_next(
        outp2, b2.reshape(1, 64), wc3, 16, 8, 8)
    outp3 = _edge_phase(h3, asad3, adas3, mv3, src_f, dst_f, 16, 16)

    out = _tc_final(outp3, b3.reshape(1, 16))
    return out[:N]
